# baseline (speedup 1.0000x reference)
import functools

import jax
import jax.numpy as jnp
from jax import lax
from jax.experimental import pallas as pl
from jax.experimental.pallas import tpu as pltpu
from jax.experimental.pallas import tpu_sc as plsc

_BT = 256
_NSLOT = 6144
_NTILES = _NSLOT // _BT
_NW = 32
_LANES = 16
_GW = 128


def _router_kernel(x_ref, wg_ref, bg_ref,
                   pos0_ref, pos1_ref, g1b_ref, g2b_ref, te_ref, ent_ref):
    n, _ = x_ref.shape
    ne = wg_ref.shape[1]
    logits = jnp.dot(x_ref[...], wg_ref[...],
                     preferred_element_type=jnp.float32) + bg_ref[...]
    idx = lax.broadcasted_iota(jnp.int32, (n, ne), 1)
    m1 = jnp.max(logits, axis=1, keepdims=True)
    i1 = jnp.min(jnp.where(logits == m1, idx, ne), axis=1, keepdims=True)
    oh1 = (idx == i1).astype(jnp.float32)
    masked = jnp.where(idx == i1, -jnp.inf, logits)
    m2 = jnp.max(masked, axis=1, keepdims=True)
    i2 = jnp.min(jnp.where(masked == m2, idx, ne), axis=1, keepdims=True)
    oh2 = (idx == i2).astype(jnp.float32)

    z = jnp.exp(m2 - m1)
    denom = 1.0 + z
    g1 = 1.0 / denom
    g2 = z / denom
    g1b_ref[...] = g1
    g2b_ref[...] = g2
    ent_tok = -(g1 * jnp.log(jnp.clip(g1, 1e-8, None))
                + g2 * jnp.log(jnp.clip(g2, 1e-8, None)))
    ent_ref[...] = (jnp.sum(ent_tok) / n).reshape(1, 1)

    ch = 256
    ri = lax.broadcasted_iota(jnp.int32, (ch, ch), 0)
    ci = lax.broadcasted_iota(jnp.int32, (ch, ch), 1)
    ltri = (ci < ri).astype(jnp.float32)
    ohcat = jnp.concatenate([oh1, oh2], axis=1)
    carry = jnp.zeros((1, 2 * ne), jnp.float32)
    chunks = []
    for c in range(n // ch):
        ohc = ohcat[c * ch:(c + 1) * ch, :]
        chunks.append(jnp.dot(ltri, ohc, preferred_element_type=jnp.float32)
                      + carry)
        carry = carry + jnp.sum(ohc, axis=0, keepdims=True)
    csum = jnp.concatenate(chunks, axis=0)
    c0 = csum[:, :ne]
    c1 = csum[:, ne:]

    cnt0 = jnp.sum(oh1, axis=0, keepdims=True)
    cnt1 = jnp.sum(oh2, axis=0, keepdims=True)
    counts = cnt0 + cnt1
    pcnt = jnp.floor((counts + (_BT - 1)) / _BT) * _BT
    eri = lax.broadcasted_iota(jnp.int32, (ne, ne), 0)
    eci = lax.broadcasted_iota(jnp.int32, (ne, ne), 1)
    estri = (eri < eci).astype(jnp.float32)
    offs = jnp.dot(pcnt, estri, preferred_element_type=jnp.float32)

    pos0 = jnp.sum(oh1 * (offs + c0), axis=1, keepdims=True)
    pos1 = jnp.sum(oh2 * (offs + cnt0 + c1), axis=1, keepdims=True)
    pos0_ref[...] = pos0.astype(jnp.int32)
    pos1_ref[...] = pos1.astype(jnp.int32)

    nt = _NTILES
    jrow = (lax.broadcasted_iota(jnp.int32, (nt, ne), 0) * _BT).astype(jnp.float32)
    offs_b = jnp.broadcast_to(offs, (nt, ne))
    te = jnp.sum((offs_b <= jrow).astype(jnp.float32), axis=1, keepdims=True) - 1.0

    te_shift = jnp.concatenate([jnp.full((1, 1), -1.0, jnp.float32),
                                te[:nt - 1]], axis=0)
    bnd = (te != te_shift).astype(jnp.float32)
    tri = lax.broadcasted_iota(jnp.int32, (nt, nt), 1) <= \
        lax.broadcasted_iota(jnp.int32, (nt, nt), 0)
    cumb = jnp.dot(tri.astype(jnp.float32), bnd,
                   preferred_element_type=jnp.float32)
    slot = (cumb - 1.0) - 2.0 * jnp.floor((cumb - 1.0) / 2.0)
    active_row = (pcnt > 0).astype(jnp.float32)
    m = (eci > eri) & (jnp.broadcast_to(active_row, (ne, ne)) > 0)
    nxt_e = jnp.min(jnp.where(m, eci.astype(jnp.float32), 99.0),
                    axis=1, keepdims=True)
    te_oh = (te == lax.broadcasted_iota(jnp.int32, (nt, ne), 1)
             .astype(jnp.float32)).astype(jnp.float32)
    nxt = jnp.dot(te_oh, nxt_e, preferred_element_type=jnp.float32)
    nxt = jnp.where(nxt == 99.0, te, nxt)

    total = jnp.sum(pcnt).reshape(1, 1)
    spx = jnp.concatenate([te, slot, bnd, nxt, total], axis=0)
    te_ref[...] = spx.astype(jnp.int32)


def _run_router(xf, Wg, bg):
    n, d = xf.shape
    ne = Wg.shape[1]
    return pl.pallas_call(
        _router_kernel,
        out_shape=[
            jax.ShapeDtypeStruct((n, 1), jnp.int32),
            jax.ShapeDtypeStruct((n, 1), jnp.int32),
            jax.ShapeDtypeStruct((n, 1), jnp.float32),
            jax.ShapeDtypeStruct((n, 1), jnp.float32),
            jax.ShapeDtypeStruct((4 * _NTILES + 1, 1), jnp.int32),
            jax.ShapeDtypeStruct((1, 1), jnp.float32),
        ],
    )(xf, Wg, bg.reshape(1, ne))


def _scatter_sc(xf, pos0, pos1):
    n, d = xf.shape
    chunk = n // _NW
    mesh = plsc.VectorSubcoreMesh(core_axis_name="c", subcore_axis_name="s")

    @functools.partial(
        pl.kernel, mesh=mesh,
        out_type=jax.ShapeDtypeStruct((_NSLOT, d), jnp.float32),
        scratch_types=[
            pltpu.VMEM((chunk, d), jnp.float32),
            pltpu.VMEM((chunk,), jnp.int32),
            pltpu.VMEM((chunk,), jnp.int32),
            pltpu.SemaphoreType.DMA,
        ],
    )
    def k(x_hbm, p0_hbm, p1_hbm, xs_hbm, xv, i0, i1, sem):
        wid = lax.axis_index("s") * 2 + lax.axis_index("c")
        base = wid * chunk
        pltpu.sync_copy(x_hbm.at[pl.ds(base, chunk)], xv)
        pltpu.sync_copy(p0_hbm.at[pl.ds(base, chunk)], i0)
        pltpu.sync_copy(p1_hbm.at[pl.ds(base, chunk)], i1)
        d1 = pltpu.async_copy(xv, xs_hbm.at[i0], sem)
        d2 = pltpu.async_copy(xv, xs_hbm.at[i1], sem)
        d1.wait()
        d2.wait()

    return k(xf, pos0, pos1)


def _ffn_kernel(sp_ref, xs_ref, w1_any, b1_ref, w2_any, b2_ref, y_ref,
                w1buf, w2buf, sem):
    j = pl.program_id(0)
    nt = _NTILES
    te = sp_ref[j]
    slot = sp_ref[nt + j]
    first = sp_ref[2 * nt + j]
    nxt = sp_ref[3 * nt + j]
    active = j * _BT < sp_ref[4 * nt]

    def w1copy(e, s):
        return pltpu.make_async_copy(w1_any.at[e], w1buf.at[s], sem)

    def w2copy(e, s):
        return pltpu.make_async_copy(w2_any.at[e], w2buf.at[s], sem)

    @pl.when(active & (j == 0))
    def _start_first():
        w1copy(te, slot).start()
        w2copy(te, slot).start()

    @pl.when(active)
    def _compute():
        @pl.when(first == 1)
        def _wait_w1():
            w1copy(te, slot).wait()

        h = jnp.dot(xs_ref[...], w1buf[slot],
                    preferred_element_type=jnp.float32)
        h = jnp.maximum(h + b1_ref[0], 0.0)

        @pl.when(first == 1)
        def _wait_w2():
            w2copy(te, slot).wait()

        @pl.when((first == 1) & (nxt != te))
        def _prefetch_next():
            w1copy(nxt, 1 - slot).start()
            w2copy(nxt, 1 - slot).start()

        y = jnp.dot(h, w2buf[slot], preferred_element_type=jnp.float32)
        y_ref[...] = y + b2_ref[0]


def _run_ffn(sp, xs, W1, b1, W2, b2):
    e, d, dff = W1.shape
    grid_spec = pltpu.PrefetchScalarGridSpec(
        num_scalar_prefetch=1,
        grid=(_NTILES,),
        in_specs=[
            pl.BlockSpec((_BT, d), lambda j, sp: (j, 0)),
            pl.BlockSpec(memory_space=pl.ANY),
            pl.BlockSpec((1, 1, dff), lambda j, sp: (sp[j], 0, 0)),
            pl.BlockSpec(memory_space=pl.ANY),
            pl.BlockSpec((1, 1, d), lambda j, sp: (sp[j], 0, 0)),
        ],
        out_specs=pl.BlockSpec((_BT, d), lambda j, sp: (j, 0)),
        scratch_shapes=[
            pltpu.VMEM((2, d, dff), jnp.float32),
            pltpu.VMEM((2, dff, d), jnp.float32),
            pltpu.SemaphoreType.DMA,
        ],
    )
    return pl.pallas_call(
        _ffn_kernel,
        grid_spec=grid_spec,
        out_shape=jax.ShapeDtypeStruct((_NSLOT, d), jnp.float32),
    )(sp, xs, W1, b1.reshape(e, 1, dff), W2, b2.reshape(e, 1, d))


def _combine_sc(y, pos0, pos1, g1, g2):
    nslot, d = y.shape
    n = pos0.shape[0]
    chunk = n // _NW
    half = chunk // 2
    mesh = plsc.VectorSubcoreMesh(core_axis_name="c", subcore_axis_name="s")

    @functools.partial(
        pl.kernel, mesh=mesh,
        out_type=jax.ShapeDtypeStruct((n, d), jnp.float32),
        scratch_types=[
            pltpu.VMEM((half, d), jnp.float32),
            pltpu.VMEM((half, d), jnp.float32),
            pltpu.VMEM((half,), jnp.int32),
            pltpu.VMEM((half,), jnp.int32),
            pltpu.VMEM((half,), jnp.float32),
            pltpu.VMEM((half,), jnp.float32),
            pltpu.SemaphoreType.DMA,
        ],
    )
    def k(y_hbm, p0_hbm, p1_hbm, g1_hbm, g2_hbm, out_hbm,
          v0, v1, i0, i1, gc0, gc1, sem):
        wid = lax.axis_index("s") * 2 + lax.axis_index("c")
        base = wid * chunk
        dn = lax.GatherDimensionNumbers(
            offset_dims=(), collapsed_slice_dims=(0,), start_index_map=(0,))
        for hh in range(2):
            hbase = base + hh * half
            pltpu.sync_copy(p0_hbm.at[pl.ds(hbase, half)], i0)
            pltpu.sync_copy(p1_hbm.at[pl.ds(hbase, half)], i1)
            pltpu.sync_copy(g1_hbm.at[pl.ds(hbase, half)], gc0)
            pltpu.sync_copy(g2_hbm.at[pl.ds(hbase, half)], gc1)
            da = pltpu.async_copy(y_hbm.at[i0], v0, sem)
            db = pltpu.async_copy(y_hbm.at[i1], v1, sem)
            da.wait()
            db.wait()

            def body(r, _):
                grp = pl.ds((r // _LANES) * _LANES, _LANES)
                bidx = jnp.full((_LANES,), r % _LANES, jnp.int32)
                s0 = lax.gather(gc0[grp], bidx[:, None], dn, (1,),
                                mode=lax.GatherScatterMode.PROMISE_IN_BOUNDS)
                s1 = lax.gather(gc1[grp], bidx[:, None], dn, (1,),
                                mode=lax.GatherScatterMode.PROMISE_IN_BOUNDS)
                for c in range(d // _LANES):
                    sl = pl.ds(c * _LANES, _LANES)
                    v0[r, sl] = v0[r, sl] * s0 + v1[r, sl] * s1
                return 0

            lax.fori_loop(0, half, body, 0)
            pltpu.sync_copy(v0, out_hbm.at[pl.ds(hbase, half)])

    return k(y, pos0, pos1, g1, g2)


def kernel(x, W1, b1, W2, b2, Wg, bg):
    B, N, D = x.shape
    E, _, DFF = W1.shape
    xf = x.reshape(N, D)

    pos0, pos1, g1, g2, te, ent = _run_router(xf, Wg, bg)
    pos0 = pos0.reshape(N)
    pos1 = pos1.reshape(N)
    xs = _scatter_sc(xf, pos0, pos1)
    y = _run_ffn(te.reshape(4 * _NTILES + 1), xs, W1, b1, W2, b2)
    out = _combine_sc(y, pos0, pos1, g1.reshape(N), g2.reshape(N))
    return out.reshape(B, N, D), ent[0, 0]


# speedup vs baseline: 1.0655x; 1.0655x over previous
import functools

import jax
import jax.numpy as jnp
from jax import lax
from jax.experimental import pallas as pl
from jax.experimental.pallas import tpu as pltpu
from jax.experimental.pallas import tpu_sc as plsc

_BT = 256
_NSLOT = 6144
_NTILES = _NSLOT // _BT
_NW = 32
_LANES = 16
_GW = 128


def _router_kernel(x_ref, wg_ref, bg_ref,
                   pos0_ref, pos1_ref, g1b_ref, g2b_ref, te_ref, ent_ref):
    n, _ = x_ref.shape
    ne = wg_ref.shape[1]
    logits = jnp.dot(x_ref[...], wg_ref[...],
                     preferred_element_type=jnp.float32) + bg_ref[...]
    idx = lax.broadcasted_iota(jnp.int32, (n, ne), 1)
    m1 = jnp.max(logits, axis=1, keepdims=True)
    i1 = jnp.min(jnp.where(logits == m1, idx, ne), axis=1, keepdims=True)
    oh1 = (idx == i1).astype(jnp.float32)
    masked = jnp.where(idx == i1, -jnp.inf, logits)
    m2 = jnp.max(masked, axis=1, keepdims=True)
    i2 = jnp.min(jnp.where(masked == m2, idx, ne), axis=1, keepdims=True)
    oh2 = (idx == i2).astype(jnp.float32)

    z = jnp.exp(m2 - m1)
    denom = 1.0 + z
    g1 = 1.0 / denom
    g2 = z / denom
    g1b_ref[...] = g1
    g2b_ref[...] = g2
    ent_tok = -(g1 * jnp.log(jnp.clip(g1, 1e-8, None))
                + g2 * jnp.log(jnp.clip(g2, 1e-8, None)))
    ent_ref[...] = (jnp.sum(ent_tok) / n).reshape(1, 1)

    ch = 256
    ri = lax.broadcasted_iota(jnp.int32, (ch, ch), 0)
    ci = lax.broadcasted_iota(jnp.int32, (ch, ch), 1)
    ltri = (ci < ri).astype(jnp.float32)
    ohcat = jnp.concatenate([oh1, oh2], axis=1)
    carry = jnp.zeros((1, 2 * ne), jnp.float32)
    chunks = []
    for c in range(n // ch):
        ohc = ohcat[c * ch:(c + 1) * ch, :]
        chunks.append(jnp.dot(ltri, ohc, preferred_element_type=jnp.float32)
                      + carry)
        carry = carry + jnp.sum(ohc, axis=0, keepdims=True)
    csum = jnp.concatenate(chunks, axis=0)
    c0 = csum[:, :ne]
    c1 = csum[:, ne:]

    cnt0 = jnp.sum(oh1, axis=0, keepdims=True)
    cnt1 = jnp.sum(oh2, axis=0, keepdims=True)
    counts = cnt0 + cnt1
    pcnt = jnp.floor((counts + (_BT - 1)) / _BT) * _BT
    eri = lax.broadcasted_iota(jnp.int32, (ne, ne), 0)
    eci = lax.broadcasted_iota(jnp.int32, (ne, ne), 1)
    estri = (eri < eci).astype(jnp.float32)
    offs = jnp.dot(pcnt, estri, preferred_element_type=jnp.float32)

    pos0 = jnp.sum(oh1 * (offs + c0), axis=1, keepdims=True)
    pos1 = jnp.sum(oh2 * (offs + cnt0 + c1), axis=1, keepdims=True)
    pos0_ref[...] = pos0.astype(jnp.int32)
    pos1_ref[...] = pos1.astype(jnp.int32)

    nt = _NTILES
    jrow = (lax.broadcasted_iota(jnp.int32, (nt, ne), 0) * _BT).astype(jnp.float32)
    offs_b = jnp.broadcast_to(offs, (nt, ne))
    te = jnp.sum((offs_b <= jrow).astype(jnp.float32), axis=1, keepdims=True) - 1.0

    te_shift = jnp.concatenate([jnp.full((1, 1), -1.0, jnp.float32),
                                te[:nt - 1]], axis=0)
    bnd = (te != te_shift).astype(jnp.float32)
    tri = lax.broadcasted_iota(jnp.int32, (nt, nt), 1) <= \
        lax.broadcasted_iota(jnp.int32, (nt, nt), 0)
    cumb = jnp.dot(tri.astype(jnp.float32), bnd,
                   preferred_element_type=jnp.float32)
    slot = (cumb - 1.0) - 2.0 * jnp.floor((cumb - 1.0) / 2.0)
    active_row = (pcnt > 0).astype(jnp.float32)
    m = (eci > eri) & (jnp.broadcast_to(active_row, (ne, ne)) > 0)
    nxt_e = jnp.min(jnp.where(m, eci.astype(jnp.float32), 99.0),
                    axis=1, keepdims=True)
    te_oh = (te == lax.broadcasted_iota(jnp.int32, (nt, ne), 1)
             .astype(jnp.float32)).astype(jnp.float32)
    nxt = jnp.dot(te_oh, nxt_e, preferred_element_type=jnp.float32)
    nxt = jnp.where(nxt == 99.0, te, nxt)

    total = jnp.sum(pcnt).reshape(1, 1)
    spx = jnp.concatenate([te, slot, bnd, nxt, total], axis=0)
    te_ref[...] = spx.astype(jnp.int32)


def _run_router(xf, Wg, bg):
    n, d = xf.shape
    ne = Wg.shape[1]
    return pl.pallas_call(
        _router_kernel,
        out_shape=[
            jax.ShapeDtypeStruct((n, 1), jnp.int32),
            jax.ShapeDtypeStruct((n, 1), jnp.int32),
            jax.ShapeDtypeStruct((n, 1), jnp.float32),
            jax.ShapeDtypeStruct((n, 1), jnp.float32),
            jax.ShapeDtypeStruct((4 * _NTILES + 1, 1), jnp.int32),
            jax.ShapeDtypeStruct((1, 1), jnp.float32),
        ],
    )(xf, Wg, bg.reshape(1, ne))


def _scatter_sc(xf, pos0, pos1):
    n, d = xf.shape
    chunk = n // _NW
    mesh = plsc.VectorSubcoreMesh(core_axis_name="c", subcore_axis_name="s")

    @functools.partial(
        pl.kernel, mesh=mesh,
        out_type=jax.ShapeDtypeStruct((_NSLOT, d), jnp.float32),
        scratch_types=[
            pltpu.VMEM((chunk, d), jnp.float32),
            pltpu.VMEM((chunk,), jnp.int32),
            pltpu.VMEM((chunk,), jnp.int32),
            pltpu.SemaphoreType.DMA,
        ],
    )
    def k(x_hbm, p0_hbm, p1_hbm, xs_hbm, xv, i0, i1, sem):
        wid = lax.axis_index("s") * 2 + lax.axis_index("c")
        base = wid * chunk
        pltpu.sync_copy(x_hbm.at[pl.ds(base, chunk)], xv)
        pltpu.sync_copy(p0_hbm.at[pl.ds(base, chunk)], i0)
        pltpu.sync_copy(p1_hbm.at[pl.ds(base, chunk)], i1)
        d1 = pltpu.async_copy(xv, xs_hbm.at[i0], sem)
        d2 = pltpu.async_copy(xv, xs_hbm.at[i1], sem)
        d1.wait()
        d2.wait()

    return k(xf, pos0, pos1)


def _ffn_kernel(sp_ref, xs_ref, w1_any, b1_ref, w2_any, b2_ref, y_ref,
                w1buf, w2buf, sem):
    j = pl.program_id(0)
    nt = _NTILES
    te = sp_ref[j]
    slot = sp_ref[nt + j]
    first = sp_ref[2 * nt + j]
    nxt = sp_ref[3 * nt + j]
    active = j * _BT < sp_ref[4 * nt]

    def w1copy(e, s):
        return pltpu.make_async_copy(w1_any.at[e], w1buf.at[s], sem)

    def w2copy(e, s):
        return pltpu.make_async_copy(w2_any.at[e], w2buf.at[s], sem)

    @pl.when(active & (j == 0))
    def _start_first():
        w1copy(te, slot).start()
        w2copy(te, slot).start()

    @pl.when(active & (first == 1))
    def _wait_current():
        w1copy(te, slot).wait()
        w2copy(te, slot).wait()

    @pl.when(active & (first == 1) & (nxt != te))
    def _prefetch_next():
        w1copy(nxt, 1 - slot).start()
        w2copy(nxt, 1 - slot).start()

    @pl.when(active)
    def _compute():
        h = jnp.dot(xs_ref[...], w1buf[slot],
                    preferred_element_type=jnp.float32)
        h = jnp.maximum(h + b1_ref[0], 0.0)
        y = jnp.dot(h, w2buf[slot], preferred_element_type=jnp.float32)
        y_ref[...] = y + b2_ref[0]


def _run_ffn(sp, xs, W1, b1, W2, b2):
    e, d, dff = W1.shape
    grid_spec = pltpu.PrefetchScalarGridSpec(
        num_scalar_prefetch=1,
        grid=(_NTILES,),
        in_specs=[
            pl.BlockSpec((_BT, d), lambda j, sp: (j, 0)),
            pl.BlockSpec(memory_space=pl.ANY),
            pl.BlockSpec((1, 1, dff), lambda j, sp: (sp[j], 0, 0)),
            pl.BlockSpec(memory_space=pl.ANY),
            pl.BlockSpec((1, 1, d), lambda j, sp: (sp[j], 0, 0)),
        ],
        out_specs=pl.BlockSpec((_BT, d), lambda j, sp: (j, 0)),
        scratch_shapes=[
            pltpu.VMEM((2, d, dff), jnp.float32),
            pltpu.VMEM((2, dff, d), jnp.float32),
            pltpu.SemaphoreType.DMA,
        ],
    )
    return pl.pallas_call(
        _ffn_kernel,
        grid_spec=grid_spec,
        out_shape=jax.ShapeDtypeStruct((_NSLOT, d), jnp.float32),
    )(sp, xs, W1, b1.reshape(e, 1, dff), W2, b2.reshape(e, 1, d))


def _combine_sc(y, pos0, pos1, g1, g2):
    nslot, d = y.shape
    n = pos0.shape[0]
    chunk = n // _NW
    half = chunk // 2
    mesh = plsc.VectorSubcoreMesh(core_axis_name="c", subcore_axis_name="s")

    @functools.partial(
        pl.kernel, mesh=mesh,
        out_type=jax.ShapeDtypeStruct((n, d), jnp.float32),
        scratch_types=[
            pltpu.VMEM((half, d), jnp.float32),
            pltpu.VMEM((half, d), jnp.float32),
            pltpu.VMEM((half,), jnp.int32),
            pltpu.VMEM((half,), jnp.int32),
            pltpu.VMEM((half,), jnp.float32),
            pltpu.VMEM((half,), jnp.float32),
            pltpu.SemaphoreType.DMA,
        ],
    )
    def k(y_hbm, p0_hbm, p1_hbm, g1_hbm, g2_hbm, out_hbm,
          v0, v1, i0, i1, gc0, gc1, sem):
        wid = lax.axis_index("s") * 2 + lax.axis_index("c")
        base = wid * chunk
        dn = lax.GatherDimensionNumbers(
            offset_dims=(), collapsed_slice_dims=(0,), start_index_map=(0,))
        for hh in range(2):
            hbase = base + hh * half
            pltpu.sync_copy(p0_hbm.at[pl.ds(hbase, half)], i0)
            pltpu.sync_copy(p1_hbm.at[pl.ds(hbase, half)], i1)
            pltpu.sync_copy(g1_hbm.at[pl.ds(hbase, half)], gc0)
            pltpu.sync_copy(g2_hbm.at[pl.ds(hbase, half)], gc1)
            da = pltpu.async_copy(y_hbm.at[i0], v0, sem)
            db = pltpu.async_copy(y_hbm.at[i1], v1, sem)
            da.wait()
            db.wait()

            def body(r, _):
                grp = pl.ds((r // _LANES) * _LANES, _LANES)
                bidx = jnp.full((_LANES,), r % _LANES, jnp.int32)
                s0 = lax.gather(gc0[grp], bidx[:, None], dn, (1,),
                                mode=lax.GatherScatterMode.PROMISE_IN_BOUNDS)
                s1 = lax.gather(gc1[grp], bidx[:, None], dn, (1,),
                                mode=lax.GatherScatterMode.PROMISE_IN_BOUNDS)
                for c in range(d // _LANES):
                    sl = pl.ds(c * _LANES, _LANES)
                    v0[r, sl] = v0[r, sl] * s0 + v1[r, sl] * s1
                return 0

            lax.fori_loop(0, half, body, 0)
            pltpu.sync_copy(v0, out_hbm.at[pl.ds(hbase, half)])

    return k(y, pos0, pos1, g1, g2)


def kernel(x, W1, b1, W2, b2, Wg, bg):
    B, N, D = x.shape
    E, _, DFF = W1.shape
    xf = x.reshape(N, D)

    pos0, pos1, g1, g2, te, ent = _run_router(xf, Wg, bg)
    pos0 = pos0.reshape(N)
    pos1 = pos1.reshape(N)
    xs = _scatter_sc(xf, pos0, pos1)
    y = _run_ffn(te.reshape(4 * _NTILES + 1), xs, W1, b1, W2, b2)
    out = _combine_sc(y, pos0, pos1, g1.reshape(N), g2.reshape(N))
    return out.reshape(B, N, D), ent[0, 0]
